# baseline (speedup 1.0000x reference)
import functools

import jax
import jax.numpy as jnp
from jax import lax
from jax.experimental import pallas as pl
from jax.experimental.pallas import tpu as pltpu
from jax.experimental.pallas import tpu_sc as plsc

_L = 16
_T_OUT = 4096
_CHUNK = 128


def _lr_body(B, N, D, T, x_hbm, dur_hbm, tl_hbm, out_hbm, mask_hbm,
             dur_v, tl_v, A_v, fidx_v, mask_v, gbuf, zbuf, gsem, osem, zsem):
  half = T // 2
  nchunk = half // _CHUNK
  wid = lax.axis_index("s") * 2 + lax.axis_index("c")
  b = wid // 2
  h = wid % 2

  pltpu.sync_copy(dur_hbm.at[b], dur_v)
  pltpu.sync_copy(tl_hbm, tl_v)
  tl_s = jnp.max(tl_v[...])

  zeros_i = jnp.zeros((_L,), jnp.int32)
  zeros_f = jnp.zeros((_L,), jnp.float32)
  iota = lax.iota(jnp.int32, _L)

  def zero_a(i, _):
    A_v[pl.ds(i * _L, _L)] = zeros_i
    return 0
  lax.fori_loop(0, T // _L, zero_a, 0)

  def zero_z(i, _):
    for v in range(D // _L):
      zbuf[i, pl.ds(v * _L, _L)] = zeros_f
    return 0
  lax.fori_loop(0, _CHUNK, zero_z, 0)

  def scan_dur(i, carry):
    v = dur_v[pl.ds(i * _L, _L)]
    s = plsc.cumsum(v) + carry
    s_c = jnp.minimum(s, tl_s)
    start = jnp.minimum(s - v, tl_s)
    m = (s_c > start) & (start < T)
    plsc.store_scatter(A_v, [jnp.minimum(start, T - 1)], i * _L + iota, mask=m)
    return jnp.max(s)
  raw_total = lax.fori_loop(0, N // _L, scan_dur, jnp.int32(0))
  total = jnp.minimum(raw_total, tl_s)

  vc = (total + _CHUNK - 1) // _CHUNK
  gv = jnp.clip((vc + 1 - h) // 2, 0, nchunk)

  def cbase(g):
    return (h + 2 * g) * _CHUNK

  def zfill(g, _):
    pltpu.async_copy(zbuf, out_hbm.at[pl.ds(b * T + cbase(g), _CHUNK)], zsem)
    return 0
  lax.fori_loop(gv, nchunk, zfill, 0)

  base = b * N
  cgroups = _CHUNK // _L

  def scan_frames(j, carry):
    a = A_v[pl.ds(j * _L, _L)]
    idxv = jnp.maximum(plsc.cummax(a), carry)
    fidx_v[pl.ds(j * _L, _L)] = base + idxv
    return jnp.max(idxv)

  def chunk_groups(g):
    return jnp.minimum((h + 2 * g + 1) * cgroups, T // _L)

  def gstart(g, p):
    pltpu.async_copy(x_hbm.at[fidx_v.at[pl.ds(cbase(g), _CHUNK)]],
                     gbuf.at[p], gsem.at[p])

  def gwait(g, p):
    pltpu.make_async_copy(x_hbm.at[fidx_v.at[pl.ds(cbase(g), _CHUNK)]],
                          gbuf.at[p], gsem.at[p]).wait()

  def ostart(g, p):
    pltpu.async_copy(gbuf.at[p], out_hbm.at[pl.ds(b * T + cbase(g), _CHUNK)],
                     osem.at[p])

  def owait(g, p):
    pltpu.make_async_copy(gbuf.at[p],
                          out_hbm.at[pl.ds(b * T + cbase(g), _CHUNK)],
                          osem.at[p]).wait()

  ptr0 = jnp.where(gv > 0, chunk_groups(0), 0)
  carry0 = lax.fori_loop(0, ptr0, scan_frames, jnp.int32(0))

  @pl.when(gv > 0)
  def _():
    gstart(0, 0)

  def pipe(g, state):
    ptr, carry = state
    p = g % 2
    q = 1 - p

    tgrp = jnp.where(g + 1 < gv, chunk_groups(g + 1), ptr)
    carry = lax.fori_loop(ptr, tgrp, scan_frames, carry)

    @pl.when(g + 1 < gv)
    def _():
      @pl.when(g >= 1)
      def _():
        owait(g - 1, q)
      gstart(g + 1, q)

    gwait(g, p)
    nvalid = jnp.clip(total - cbase(g), 0, _CHUNK)

    def zrow(r, _):
      for v in range(D // _L):
        gbuf[p, r, pl.ds(v * _L, _L)] = zeros_f
      return 0
    lax.fori_loop(nvalid, _CHUNK, zrow, 0)
    ostart(g, p)
    return (tgrp, carry)
  lax.fori_loop(0, gv, pipe, (ptr0, carry0))

  def mrow(j, _):
    tvec = h * half + j * _L + iota
    mask_v[pl.ds(j * _L, _L)] = (tvec < total).astype(jnp.int32)
    return 0
  lax.fori_loop(0, half // _L, mrow, 0)
  pltpu.sync_copy(mask_v, mask_hbm.at[pl.ds(b * T + h * half, half)])

  @pl.when(gv >= 2)
  def _():
    owait(gv - 2, gv % 2)

  @pl.when(gv >= 1)
  def _():
    owait(gv - 1, (gv + 1) % 2)

  def zdrain(i, _):
    pltpu.make_async_copy(zbuf, out_hbm.at[pl.ds(b * T + cbase(gv + i), _CHUNK)],
                          zsem).wait()
    return 0
  lax.fori_loop(0, nchunk - gv, zdrain, 0)


def kernel(x, durations, target_len):
  B, N, D = x.shape
  T = _T_OUT
  tl_arr = jnp.full((_L,), target_len, jnp.int32)

  mesh = plsc.VectorSubcoreMesh(core_axis_name="c", subcore_axis_name="s")
  out_flat, mask_flat = pl.kernel(
      functools.partial(_lr_body, B, N, D, T),
      out_type=(jax.ShapeDtypeStruct((B * T, D), jnp.float32),
                jax.ShapeDtypeStruct((B * T,), jnp.int32)),
      mesh=mesh,
      compiler_params=pltpu.CompilerParams(needs_layout_passes=False),
      scratch_types=[
          pltpu.VMEM((N,), jnp.int32),
          pltpu.VMEM((_L,), jnp.int32),
          pltpu.VMEM((T,), jnp.int32),
          pltpu.VMEM((T,), jnp.int32),
          pltpu.VMEM((T // 2,), jnp.int32),
          pltpu.VMEM((2, _CHUNK, D), jnp.float32),
          pltpu.VMEM((_CHUNK, D), jnp.float32),
          pltpu.SemaphoreType.DMA((2,)),
          pltpu.SemaphoreType.DMA((2,)),
          pltpu.SemaphoreType.DMA,
      ],
  )(x.reshape(B * N, D), durations.astype(jnp.int32), tl_arr)
  return out_flat.reshape(B, T, D), (mask_flat.reshape(B, T) != 0)


# speedup vs baseline: 1.0259x; 1.0259x over previous
import functools

import jax
import jax.numpy as jnp
from jax import lax
from jax.experimental import pallas as pl
from jax.experimental.pallas import tpu as pltpu
from jax.experimental.pallas import tpu_sc as plsc

_L = 16
_T_OUT = 4096
_CHUNK = 128


def _lr_body(B, N, D, T, x_hbm, dur_hbm, tl_hbm, out_hbm, mask_hbm,
             dur_v, tl_v, A_v, fidx_v, mask_v, gbuf, zbuf, gsem, osem, zsem):
  half = T // 2
  nchunk = half // _CHUNK
  wid = lax.axis_index("s") * 2 + lax.axis_index("c")
  b = wid // 2
  h = wid % 2

  pltpu.sync_copy(dur_hbm.at[b], dur_v)
  pltpu.sync_copy(tl_hbm, tl_v)
  tl_s = jnp.max(tl_v[...])

  zeros_i = jnp.zeros((_L,), jnp.int32)
  zeros_f = jnp.zeros((_L,), jnp.float32)
  iota = lax.iota(jnp.int32, _L)

  def zero_a(i, _):
    A_v[pl.ds(i * _L, _L)] = zeros_i
    return 0
  lax.fori_loop(0, T // _L, zero_a, 0)

  def zero_z(i, _):
    for v in range(D // _L):
      zbuf[i, pl.ds(v * _L, _L)] = zeros_f
    return 0
  lax.fori_loop(0, _CHUNK, zero_z, 0)

  def scan_dur(i, carry):
    v = dur_v[pl.ds(i * _L, _L)]
    s = plsc.cumsum(v) + carry
    s_c = jnp.minimum(s, tl_s)
    start = jnp.minimum(s - v, tl_s)
    m = (s_c > start) & (start < T)
    plsc.store_scatter(A_v, [jnp.minimum(start, T - 1)], i * _L + iota, mask=m)
    return jnp.max(s)
  raw_total = lax.fori_loop(0, N // _L, scan_dur, jnp.int32(0))
  total = jnp.minimum(raw_total, tl_s)

  vc = (total + _CHUNK - 1) // _CHUNK
  gv = jnp.clip((vc + 1 - h) // 2, 0, nchunk)

  def cbase(g):
    return (h + 2 * g) * _CHUNK

  def zfill(g, _):
    pltpu.async_copy(zbuf, out_hbm.at[pl.ds(b * T + cbase(g), _CHUNK)], zsem)
    return 0
  lax.fori_loop(gv, nchunk, zfill, 0)

  base = b * N
  cgroups = _CHUNK // _L

  def scan_frames(j, carry):
    a = A_v[pl.ds(j * _L, _L)]
    idxv = jnp.maximum(plsc.cummax(a), carry)
    fidx_v[pl.ds(j * _L, _L)] = base + idxv
    return jnp.max(idxv)

  def chunk_groups(g):
    return jnp.minimum((h + 2 * g + 1) * cgroups, T // _L)

  def gstart(g, p):
    pltpu.async_copy(x_hbm.at[fidx_v.at[pl.ds(cbase(g), _CHUNK)]],
                     gbuf.at[p], gsem.at[p])

  def gwait(g, p):
    pltpu.make_async_copy(x_hbm.at[fidx_v.at[pl.ds(cbase(g), _CHUNK)]],
                          gbuf.at[p], gsem.at[p]).wait()

  def ostart(g, p):
    pltpu.async_copy(gbuf.at[p], out_hbm.at[pl.ds(b * T + cbase(g), _CHUNK)],
                     osem.at[p])

  def owait(g, p):
    pltpu.make_async_copy(gbuf.at[p],
                          out_hbm.at[pl.ds(b * T + cbase(g), _CHUNK)],
                          osem.at[p]).wait()

  ptr0 = jnp.where(gv > 0, chunk_groups(jnp.minimum(1, gv - 1)), 0)
  carry0 = lax.fori_loop(0, ptr0, scan_frames, jnp.int32(0))

  @pl.when(gv > 0)
  def _():
    gstart(0, 0)

  def pipe(g, state):
    ptr, carry = state
    p = g % 2
    q = 1 - p

    @pl.when(g + 1 < gv)
    def _():
      @pl.when(g >= 1)
      def _():
        owait(g - 1, q)
      gstart(g + 1, q)

    tgrp = jnp.where(g + 2 < gv, chunk_groups(g + 2), ptr)
    carry = lax.fori_loop(ptr, tgrp, scan_frames, carry)

    gwait(g, p)
    nvalid = jnp.clip(total - cbase(g), 0, _CHUNK)

    def zrow(r, _):
      for v in range(D // _L):
        gbuf[p, r, pl.ds(v * _L, _L)] = zeros_f
      return 0
    lax.fori_loop(nvalid, _CHUNK, zrow, 0)
    ostart(g, p)
    return (tgrp, carry)
  lax.fori_loop(0, gv, pipe, (ptr0, carry0))

  def mrow(j, _):
    tvec = h * half + j * _L + iota
    mask_v[pl.ds(j * _L, _L)] = (tvec < total).astype(jnp.int32)
    return 0
  lax.fori_loop(0, half // _L, mrow, 0)
  pltpu.sync_copy(mask_v, mask_hbm.at[pl.ds(b * T + h * half, half)])

  @pl.when(gv >= 2)
  def _():
    owait(gv - 2, gv % 2)

  @pl.when(gv >= 1)
  def _():
    owait(gv - 1, (gv + 1) % 2)

  def zdrain(i, _):
    pltpu.make_async_copy(zbuf, out_hbm.at[pl.ds(b * T + cbase(gv + i), _CHUNK)],
                          zsem).wait()
    return 0
  lax.fori_loop(0, nchunk - gv, zdrain, 0)


def kernel(x, durations, target_len):
  B, N, D = x.shape
  T = _T_OUT
  tl_arr = jnp.full((_L,), target_len, jnp.int32)

  mesh = plsc.VectorSubcoreMesh(core_axis_name="c", subcore_axis_name="s")
  out_flat, mask_flat = pl.kernel(
      functools.partial(_lr_body, B, N, D, T),
      out_type=(jax.ShapeDtypeStruct((B * T, D), jnp.float32),
                jax.ShapeDtypeStruct((B * T,), jnp.int32)),
      mesh=mesh,
      compiler_params=pltpu.CompilerParams(needs_layout_passes=False),
      scratch_types=[
          pltpu.VMEM((N,), jnp.int32),
          pltpu.VMEM((_L,), jnp.int32),
          pltpu.VMEM((T,), jnp.int32),
          pltpu.VMEM((T,), jnp.int32),
          pltpu.VMEM((T // 2,), jnp.int32),
          pltpu.VMEM((2, _CHUNK, D), jnp.float32),
          pltpu.VMEM((_CHUNK, D), jnp.float32),
          pltpu.SemaphoreType.DMA((2,)),
          pltpu.SemaphoreType.DMA((2,)),
          pltpu.SemaphoreType.DMA,
      ],
  )(x.reshape(B * N, D), durations.astype(jnp.int32), tl_arr)
  return out_flat.reshape(B, T, D), (mask_flat.reshape(B, T) != 0)


# speedup vs baseline: 1.0387x; 1.0125x over previous
import functools

import jax
import jax.numpy as jnp
from jax import lax
from jax.experimental import pallas as pl
from jax.experimental.pallas import tpu as pltpu
from jax.experimental.pallas import tpu_sc as plsc

_L = 16
_T_OUT = 4096
_CHUNK = 128
_NBUF = 3
_ZROWS = 32


def _lr_body(B, N, D, T, x_hbm, dur_hbm, tl_hbm, out_hbm, mask_hbm,
             dur_v, tl_v, A_v, fidx_v, mask_v, gbuf, zbuf, gsem, osem, zsem):
  half = T // 2
  nchunk = half // _CHUNK
  wid = lax.axis_index("s") * 2 + lax.axis_index("c")
  b = wid // 2
  h = wid % 2

  pltpu.sync_copy(dur_hbm.at[b], dur_v)
  pltpu.sync_copy(tl_hbm, tl_v)
  tl_s = jnp.max(tl_v[...])

  zeros_i = jnp.zeros((_L,), jnp.int32)
  zeros_f = jnp.zeros((_L,), jnp.float32)
  iota = lax.iota(jnp.int32, _L)

  def zero_a(i, _):
    A_v[pl.ds(i * _L, _L)] = zeros_i
    return 0
  lax.fori_loop(0, T // _L, zero_a, 0)

  def zero_z(i, _):
    for v in range(D // _L):
      zbuf[i, pl.ds(v * _L, _L)] = zeros_f
    return 0
  lax.fori_loop(0, _ZROWS, zero_z, 0)

  def scan_dur(i, carry):
    v = dur_v[pl.ds(i * _L, _L)]
    s = plsc.cumsum(v) + carry
    s_c = jnp.minimum(s, tl_s)
    start = jnp.minimum(s - v, tl_s)
    m = (s_c > start) & (start < T)
    plsc.store_scatter(A_v, [jnp.minimum(start, T - 1)], i * _L + iota, mask=m)
    return jnp.max(s)
  raw_total = lax.fori_loop(0, N // _L, scan_dur, jnp.int32(0))
  total = jnp.minimum(raw_total, tl_s)

  vc = (total + _CHUNK - 1) // _CHUNK
  gv = jnp.clip((vc + 1 - h) // 2, 0, nchunk)

  def cbase(g):
    return (h + 2 * g) * _CHUNK

  def zfill(g, _):
    for sub in range(_CHUNK // _ZROWS):
      pltpu.async_copy(
          zbuf, out_hbm.at[pl.ds(b * T + cbase(g) + sub * _ZROWS, _ZROWS)],
          zsem)
    return 0
  lax.fori_loop(gv, nchunk, zfill, 0)

  base = b * N
  cgroups = _CHUNK // _L

  def scan_frames(j, carry):
    a = A_v[pl.ds(j * _L, _L)]
    idxv = jnp.maximum(plsc.cummax(a), carry)
    fidx_v[pl.ds(j * _L, _L)] = base + idxv
    return jnp.max(idxv)

  def chunk_groups(g):
    return jnp.minimum((h + 2 * g + 1) * cgroups, T // _L)

  def gstart(g, p):
    pltpu.async_copy(x_hbm.at[fidx_v.at[pl.ds(cbase(g), _CHUNK)]],
                     gbuf.at[p], gsem.at[p])

  def gwait(g, p):
    pltpu.make_async_copy(x_hbm.at[fidx_v.at[pl.ds(cbase(g), _CHUNK)]],
                          gbuf.at[p], gsem.at[p]).wait()

  def ostart(g, p):
    pltpu.async_copy(gbuf.at[p], out_hbm.at[pl.ds(b * T + cbase(g), _CHUNK)],
                     osem.at[p])

  def owait(g, p):
    pltpu.make_async_copy(gbuf.at[p],
                          out_hbm.at[pl.ds(b * T + cbase(g), _CHUNK)],
                          osem.at[p]).wait()

  ptr0 = jnp.where(gv > 0, chunk_groups(jnp.minimum(1, gv - 1)), 0)
  carry0 = lax.fori_loop(0, ptr0, scan_frames, jnp.int32(0))

  @pl.when(gv > 0)
  def _():
    gstart(0, 0)

  def pipe(g, state):
    ptr, carry = state
    p = g % _NBUF

    @pl.when(g + 1 < gv)
    def _():
      q = (g + 1) % _NBUF
      @pl.when(g >= _NBUF - 1)
      def _():
        owait(g + 1 - _NBUF, q)
      gstart(g + 1, q)

    tgrp = jnp.where(g + 2 < gv, chunk_groups(g + 2), ptr)
    carry = lax.fori_loop(ptr, tgrp, scan_frames, carry)

    gwait(g, p)
    nvalid = jnp.clip(total - cbase(g), 0, _CHUNK)

    def zrow(r, _):
      for v in range(D // _L):
        gbuf[p, r, pl.ds(v * _L, _L)] = zeros_f
      return 0
    lax.fori_loop(nvalid, _CHUNK, zrow, 0)
    ostart(g, p)
    return (tgrp, carry)
  lax.fori_loop(0, gv, pipe, (ptr0, carry0))

  def mrow(j, _):
    tvec = h * half + j * _L + iota
    mask_v[pl.ds(j * _L, _L)] = (tvec < total).astype(jnp.int32)
    return 0
  lax.fori_loop(0, half // _L, mrow, 0)
  pltpu.sync_copy(mask_v, mask_hbm.at[pl.ds(b * T + h * half, half)])

  for k in range(_NBUF, 0, -1):
    @pl.when(gv >= k)
    def _(k=k):
      owait(gv - k, (gv - k) % _NBUF)

  def zdrain(i, _):
    for sub in range(_CHUNK // _ZROWS):
      pltpu.make_async_copy(
          zbuf, out_hbm.at[pl.ds(b * T + cbase(gv + i) + sub * _ZROWS, _ZROWS)],
          zsem).wait()
    return 0
  lax.fori_loop(0, nchunk - gv, zdrain, 0)


def kernel(x, durations, target_len):
  B, N, D = x.shape
  T = _T_OUT
  tl_arr = jnp.full((_L,), target_len, jnp.int32)

  mesh = plsc.VectorSubcoreMesh(core_axis_name="c", subcore_axis_name="s")
  out_flat, mask_flat = pl.kernel(
      functools.partial(_lr_body, B, N, D, T),
      out_type=(jax.ShapeDtypeStruct((B * T, D), jnp.float32),
                jax.ShapeDtypeStruct((B * T,), jnp.int32)),
      mesh=mesh,
      compiler_params=pltpu.CompilerParams(needs_layout_passes=False),
      scratch_types=[
          pltpu.VMEM((N,), jnp.int32),
          pltpu.VMEM((_L,), jnp.int32),
          pltpu.VMEM((T,), jnp.int32),
          pltpu.VMEM((T,), jnp.int32),
          pltpu.VMEM((T // 2,), jnp.int32),
          pltpu.VMEM((_NBUF, _CHUNK, D), jnp.float32),
          pltpu.VMEM((_ZROWS, D), jnp.float32),
          pltpu.SemaphoreType.DMA((_NBUF,)),
          pltpu.SemaphoreType.DMA((_NBUF,)),
          pltpu.SemaphoreType.DMA,
      ],
  )(x.reshape(B * N, D), durations.astype(jnp.int32), tl_arr)
  return out_flat.reshape(B, T, D), (mask_flat.reshape(B, T) != 0)
